# Initial kernel scaffold; baseline (speedup 1.0000x reference)
import jax
import jax.numpy as jnp
from jax.experimental import pallas as pl


def kernel(input_ids, token_values, seq_lens):
    raise NotImplementedError("write your pallas kernel here")


# speedup vs baseline: 173.5384x; 173.5384x over previous
import jax
import jax.numpy as jnp
from jax.experimental import pallas as pl

_VOCAB = 32000
_COEFFS = (0.1, 0.2, 0.3, 0.4)


def _body(ids_ref, vals_ref, seq_ref, out_ref):
    B = seq_ref.shape[1]
    T = ids_ref.shape[1]
    tri = (
        jax.lax.broadcasted_iota(jnp.int32, (B, B), 0)
        >= jax.lax.broadcasted_iota(jnp.int32, (B, B), 1)
    )
    seq = jnp.broadcast_to(seq_ref[...], (B, B))
    last = jnp.sum(jnp.where(tri, seq, 0), axis=1, keepdims=True) - 1

    tidx = jax.lax.broadcasted_iota(jnp.int32, (B, T), 1)
    match = tidx == last
    sel_ids = jnp.sum(jnp.where(match, jnp.broadcast_to(ids_ref[...], (B, T)), 0),
                      axis=1, keepdims=True)
    sel_vals = jnp.sum(jnp.where(match, jnp.broadcast_to(vals_ref[...], (B, T)), 0.0),
                       axis=1, keepdims=True)

    col = jax.lax.broadcasted_iota(jnp.int32, (B, _VOCAB), 1)
    acc = jnp.zeros((B, _VOCAB), jnp.float32)
    for k, c in enumerate(_COEFFS):
        ck = (sel_ids + k) % _VOCAB
        acc = acc + jnp.where(col == ck, c * sel_vals, 0.0)
    out_ref[...] = acc


def kernel(input_ids, token_values, seq_lens):
    T = input_ids.shape[0]
    B = seq_lens.shape[0]
    return pl.pallas_call(
        _body,
        out_shape=jax.ShapeDtypeStruct((B, _VOCAB), jnp.float32),
    )(
        input_ids.reshape(1, T),
        token_values.reshape(1, T),
        seq_lens.astype(jnp.int32).reshape(1, B),
    )
